# baseline (speedup 1.0000x reference)
import jax
import jax.numpy as jnp
from jax.experimental import pallas as pl
from jax.experimental.pallas import tpu as pltpu

_N, _H, _W = 4, 64, 64
_M = 2 * _N
_BIG = 1 << 22


def _haus_kernel(packed_ref, out_ref, ht_ref):
    k_i = jax.lax.broadcasted_iota(jnp.int32, (_W, _W), 0)
    k_j = jax.lax.broadcasted_iota(jnp.int32, (_W, _W), 1)
    d2 = (k_i - k_j) * (k_i - k_j)

    packed = packed_ref[:, :]
    masks = (jnp.round(packed) > 0.5).astype(jnp.int32)
    cost_cat = jnp.where(masks > 0, jnp.int32(0), jnp.int32(_BIG))

    h_cat = jnp.min(cost_cat[:, None, :] + d2[:, :, None], axis=0)

    for p in range(_N):
        pair = h_cat[:, 2 * p * _W:(2 * p + 2) * _W]
        ht_ref[:, 2 * p * _W:(2 * p + 2) * _W] = jnp.transpose(
            pair.reshape(_W, 2, _W), (2, 1, 0)).reshape(_W, 2 * _W)

    d2t_cat = jnp.min(ht_ref[:, :][:, None, :] + d2[:, :, None], axis=0)

    total = jnp.float32(0.0)
    for i in range(_N):
        mpair = jnp.transpose(
            masks[:, 2 * i * _W:(2 * i + 2) * _W].reshape(_W, 2, _W),
            (2, 1, 0)).reshape(_W, 2 * _W)
        b_t = mpair[:, :_W] > 0
        a_t = mpair[:, _W:] > 0
        dists = []
        for q, m in ((a_t & ~b_t, 2 * i), (b_t & ~a_t, 2 * i + 1)):
            blk = d2t_cat[:, m * _W:(m + 1) * _W]
            mx = jnp.max(jnp.where(q, blk, jnp.int32(-1)))
            dist = jnp.where(mx >= _BIG, jnp.float32(jnp.inf),
                             jnp.sqrt(mx.astype(jnp.float32)) / jnp.float32(_W))
            dists.append(jnp.where(mx >= 0, dist, jnp.float32(0.0)))
        total = total + jnp.maximum(dists[0], dists[1])
    out_ref[:, :] = jnp.broadcast_to(total / jnp.float32(_N), (1, 1))


@jax.jit
def kernel(predict, target):
    p = predict.reshape(_N, _H, _W)
    t = target.reshape(_N, _H, _W)
    packed = jnp.stack([t, p], axis=1).transpose(2, 0, 1, 3).reshape(_H, _M * _W)
    out = pl.pallas_call(
        _haus_kernel,
        out_shape=jax.ShapeDtypeStruct((1, 1), jnp.float32),
        scratch_shapes=[pltpu.VMEM((_W, _M * _W), jnp.int32)],
    )(packed)
    return out[0, 0]


# speedup vs baseline: 3.6624x; 3.6624x over previous
import jax
import jax.numpy as jnp
from jax.experimental import pallas as pl
from jax.experimental.pallas import tpu as pltpu

_N, _H, _W = 4, 64, 64
_BIG = float(1 << 24)


def _haus_kernel(pred_ref, targ_ref, out_ref):
    k_i = jax.lax.broadcasted_iota(jnp.int32, (_W, _W), 0)
    k_j = jax.lax.broadcasted_iota(jnp.int32, (_W, _W), 1)
    d2 = ((k_i - k_j) * (k_i - k_j)).astype(jnp.float32)

    def minplus(cost):
        acc = cost[0:1, :] + d2[:, 0:1]
        for j in range(1, _W):
            acc = jnp.minimum(acc, cost[j:j + 1, :] + d2[:, j:j + 1])
        return acc

    def edt2(src):
        cost = jnp.where(src, jnp.float32(0.0), jnp.float32(_BIG))
        g = minplus(cost)
        return minplus(g.T)

    total = jnp.float32(0.0)
    for i in range(_N):
        a = jnp.round(pred_ref[i]) > 0.5
        b = jnp.round(targ_ref[i]) > 0.5
        dists = []
        for q_t, s in (((a & ~b).astype(jnp.int32).T, b),
                       ((b & ~a).astype(jnp.int32).T, a)):
            dd = edt2(s)
            mx = jnp.max(jnp.where(q_t > 0, dd, jnp.float32(-1.0)))
            dist = jnp.where(mx >= _BIG, jnp.float32(jnp.inf),
                             jnp.sqrt(mx) / jnp.float32(_W))
            dists.append(jnp.where(mx >= 0, dist, jnp.float32(0.0)))
        total = total + jnp.maximum(dists[0], dists[1])
    out_ref[:, :] = jnp.broadcast_to(total / jnp.float32(_N), (1, 1))


@jax.jit
def kernel(predict, target):
    p = predict.reshape(_N, _H, _W)
    t = target.reshape(_N, _H, _W)
    out = pl.pallas_call(
        _haus_kernel,
        out_shape=jax.ShapeDtypeStruct((1, 1), jnp.float32),
    )(p, t)
    return out[0, 0]


# speedup vs baseline: 4.4151x; 1.2055x over previous
import jax
import jax.numpy as jnp
from jax.experimental import pallas as pl
from jax.experimental.pallas import tpu as pltpu

_N, _H, _W = 4, 64, 64
_M = 2 * _N
_BIG = float(1 << 24)


def _haus_kernel(pred_ref, targ_ref, out_ref, st_ref):
    k_i = jax.lax.broadcasted_iota(jnp.int32, (_W, _W), 0)
    k_j = jax.lax.broadcasted_iota(jnp.int32, (_W, _W), 1)
    d2 = ((k_i - k_j) * (k_i - k_j)).astype(jnp.float32)

    def minplus_all(ct):
        acc = ct[0:1, :] + d2[:, 0:1]
        for j in range(1, _W):
            acc = jnp.minimum(acc, ct[j:j + 1, :] + d2[:, j:j + 1])
        return acc

    masks = []
    for i in range(_N):
        a = jnp.round(pred_ref[i]) > 0.5
        b = jnp.round(targ_ref[i]) > 0.5
        masks.extend([b, a])
    for m in range(_M):
        st_ref[m * _W:(m + 1) * _W, :] = jnp.where(
            masks[m], jnp.float32(0.0), jnp.float32(_BIG))

    g_all = minplus_all(st_ref[:, :].T)

    for m in range(_M):
        st_ref[m * _W:(m + 1) * _W, :] = g_all[:, m * _W:(m + 1) * _W]
    dd_all = minplus_all(st_ref[:, :].T)

    total = jnp.float32(0.0)
    for i in range(_N):
        b = masks[2 * i]
        a = masks[2 * i + 1]
        dists = []
        for q, m in ((a & ~b, 2 * i), (b & ~a, 2 * i + 1)):
            mx = jnp.max(jnp.where(q, dd_all[:, m * _W:(m + 1) * _W],
                                   jnp.float32(-1.0)))
            dist = jnp.where(mx >= _BIG, jnp.float32(jnp.inf),
                             jnp.sqrt(mx) / jnp.float32(_W))
            dists.append(jnp.where(mx >= 0, dist, jnp.float32(0.0)))
        total = total + jnp.maximum(dists[0], dists[1])
    out_ref[:, :] = jnp.broadcast_to(total / jnp.float32(_N), (1, 1))


@jax.jit
def kernel(predict, target):
    p = predict.reshape(_N, _H, _W)
    t = target.reshape(_N, _H, _W)
    out = pl.pallas_call(
        _haus_kernel,
        out_shape=jax.ShapeDtypeStruct((1, 1), jnp.float32),
        scratch_shapes=[pltpu.VMEM((_M * _W, _W), jnp.float32)],
    )(p, t)
    return out[0, 0]
